# baseline (speedup 1.0000x reference)
import functools

import jax
import jax.numpy as jnp
from jax import lax
from jax.experimental import pallas as pl
from jax.experimental.pallas import tpu as pltpu
from jax.experimental.pallas import tpu_sc as plsc

_NW = 32
_L = 16


def _sc_dots_body(ia, ib, isa, isb, ism, mka_h, mkb_h,
                  ent_in, ent_out, small, out,
                  va_i, vb_i, vsa_i, vsb_i, vsm_i, mka, mkb,
                  ba, bb, bsa, bsb, bsm, dots,
                  s0, s1, s2, s3, s4, *, ch, emb):
    wid = lax.axis_index("s") * 2 + lax.axis_index("c")
    pltpu.sync_copy(ia.at[wid], va_i)
    pltpu.sync_copy(ib.at[wid], vb_i)
    pltpu.sync_copy(isa.at[wid], vsa_i)
    pltpu.sync_copy(isb.at[wid], vsb_i)
    pltpu.sync_copy(ism.at[wid], vsm_i)
    pltpu.sync_copy(mka_h.at[wid], mka)
    pltpu.sync_copy(mkb_h.at[wid], mkb)
    cs = [pltpu.async_copy(ent_in.at[va_i], ba, s0),
          pltpu.async_copy(ent_out.at[vb_i], bb, s1),
          pltpu.async_copy(small.at[vsa_i], bsa, s2),
          pltpu.async_copy(small.at[vsb_i], bsb, s3),
          pltpu.async_copy(small.at[vsm_i], bsm, s4)]
    for c in cs:
        c.wait()
    zero = jnp.zeros((_L,), jnp.float32)
    for g in range(ch // _L):
        gsl = pl.ds(g * _L, _L)
        ma = mka[gsl]
        mb = mkb[gsl]
        pab, pam, pbm = zero, zero, zero
        for d in range(emb):
            dsl = pl.ds(d * ch + g * _L, _L)
            a = ba[dsl] * ma + bsa[dsl]
            b = bb[dsl] * mb + bsb[dsl]
            m = bsm[dsl]
            pab = pab + a * b
            pam = pam + a * m
            pbm = pbm + b * m
        dots[gsl] = pab - pam * pbm
    base = wid * ch
    pltpu.sync_copy(dots, out.at[pl.ds(base, ch)])


def _make_sc_dots(n, emb):
    ch = n // _NW
    mesh = plsc.VectorSubcoreMesh(core_axis_name="c", subcore_axis_name="s")
    return pl.kernel(
        functools.partial(_sc_dots_body, ch=ch, emb=emb),
        mesh=mesh,
        out_type=jax.ShapeDtypeStruct((n,), jnp.float32),
        scratch_types=[pltpu.VMEM((ch * emb,), jnp.int32)] * 5
        + [pltpu.VMEM((ch,), jnp.float32)] * 2
        + [pltpu.VMEM((ch * emb,), jnp.float32)] * 5
        + [pltpu.VMEM((ch,), jnp.float32)]
        + [pltpu.SemaphoreType.DMA] * 5,
        compiler_params=pltpu.CompilerParams(needs_layout_passes=False),
    )


def _log_sigmoid(x):
    return jnp.minimum(x, 0.0) - jnp.log(1.0 + jnp.exp(-jnp.abs(x)))


def _tc_loss_body(dp, dn, out):
    acc = _log_sigmoid(dp[...]) + jnp.sum(_log_sigmoid(-dn[...]),
                                          axis=1, keepdims=True)
    out[...] = -acc


def _prep(labels_in, labels_tgt, ne, rel):
    ie = labels_in < ne
    te = labels_tgt < ne
    io = jnp.where(ie, labels_in, labels_in - ne).astype(jnp.int32)
    to = jnp.where(te, labels_tgt, labels_tgt - ne).astype(jnp.int32)
    case = jnp.where(ie & te, 0, jnp.where(ie & (~te), 1,
                     jnp.where((~ie) & te, 2, 3)))
    perm = jnp.argsort(case)
    io, to, ie, te, case = io[perm], to[perm], ie[perm], te[perm], case[perm]
    zr = 4 * rel
    r_ae = jnp.where(ie, io, 0)
    r_be = jnp.where(te, to, 0)
    r_sa = jnp.where(ie, zr, io)
    r_sb = jnp.where(te, zr, rel + to)
    m_act = (case == 1) | (case == 2)
    r_sm = jnp.where(m_act, jnp.where(case == 1, 2 * rel + to, 3 * rel + io),
                     zr)
    return (r_ae, r_be, r_sa, r_sb, r_sm,
            ie.astype(jnp.float32), te.astype(jnp.float32))


def _dmajor_idx(rows, stride, nw, emb, ch):
    r = rows.reshape(nw, 1, ch)
    d = (jnp.arange(emb, dtype=jnp.int32) * stride).reshape(1, emb, 1)
    return (r + d).reshape(nw, emb * ch)


def kernel(input_labels, pos_labels, neg_labels, ent_dic, reverse_dictionary,
           in_embed_ent, out_embed_ent, in_embed_rel, out_embed_rel,
           in_embed_map, out_embed_map):
    b = input_labels.shape[0]
    k = neg_labels.shape[0] // b
    emb = in_embed_ent.shape[1]
    ne = ent_dic.shape[0]
    rel = in_embed_rel.shape[0]
    nent = in_embed_ent.shape[0]
    n = b * (k + 1)
    ch = n // _NW

    p = _prep(input_labels.reshape(-1), pos_labels.reshape(-1), ne, rel)
    q = _prep(jnp.repeat(input_labels.reshape(-1), k),
              neg_labels.reshape(-1), ne, rel)
    r_ae, r_be, r_sa, r_sb, r_sm, mka, mkb = (
        jnp.concatenate([pi, qi]) for pi, qi in zip(p, q))

    nsmall = 4 * rel + 1
    ia = _dmajor_idx(r_ae, nent, _NW, emb, ch)
    ib = _dmajor_idx(r_be, nent, _NW, emb, ch)
    isa = _dmajor_idx(r_sa, nsmall, _NW, emb, ch)
    isb = _dmajor_idx(r_sb, nsmall, _NW, emb, ch)
    ism = _dmajor_idx(r_sm, nsmall, _NW, emb, ch)

    ent_in = in_embed_ent.T.reshape(-1)
    ent_out = out_embed_ent.T.reshape(-1)
    small = jnp.concatenate(
        [in_embed_rel, out_embed_rel, in_embed_map, out_embed_map,
         jnp.zeros((1, emb), jnp.float32)], axis=0).T.reshape(-1)

    dots = _make_sc_dots(n, emb)(
        ia, ib, isa, isb, ism,
        mka.reshape(_NW, ch), mkb.reshape(_NW, ch),
        ent_in, ent_out, small)

    loss2d = pl.pallas_call(
        _tc_loss_body,
        out_shape=jax.ShapeDtypeStruct((b, 1), jnp.float32),
    )(dots[:b].reshape(b, 1), dots[b:].reshape(b, k))
    return loss2d.reshape(b)


# speedup vs baseline: 3.3922x; 3.3922x over previous
import functools

import jax
import jax.numpy as jnp
from jax import lax
from jax.experimental import pallas as pl
from jax.experimental.pallas import tpu as pltpu
from jax.experimental.pallas import tpu_sc as plsc

_NW = 32
_PASS = 96
_L = 16


def _sc_dots_body(gae, gas, gbe, gbs, gm, ra_h, rb_h, ca_h, cb_h, cm_h,
                  ent_in, ent_out, small, out,
                  a0, a1, a2, sa0, sa1, sa2, b0, b1, b2, sb0, sb1, sb2,
                  m0, m1, m2,
                  vra, vrb, vca, vcb, vcm,
                  bufa, bufb, bufm, dots,
                  s0, s1, s2, s3, s4, *, ch, emb):
    npass = ch // _PASS
    ia_r = (a0, a1, a2)
    isa_r = (sa0, sa1, sa2)
    ib_r = (b0, b1, b2)
    isb_r = (sb0, sb1, sb2)
    im_r = (m0, m1, m2)
    wid = lax.axis_index("s") * 2 + lax.axis_index("c")
    base = wid * ch
    sl_all = pl.ds(base, ch)
    for p in range(npass):
        row = wid * npass + p
        pltpu.sync_copy(gae.at[row], ia_r[p])
        pltpu.sync_copy(gas.at[row], isa_r[p])
        pltpu.sync_copy(gbe.at[row], ib_r[p])
        pltpu.sync_copy(gbs.at[row], isb_r[p])
        pltpu.sync_copy(gm.at[row], im_r[p])
    pltpu.sync_copy(ra_h.at[sl_all], vra)
    pltpu.sync_copy(rb_h.at[sl_all], vrb)
    pltpu.sync_copy(ca_h.at[sl_all], vca)
    pltpu.sync_copy(cb_h.at[sl_all], vcb)
    pltpu.sync_copy(cm_h.at[sl_all], vcm)
    zero = jnp.zeros((_L,), jnp.float32)
    for p in range(npass):
        cs = [pltpu.async_copy(ent_in.at[ia_r[p]], bufa.at[pl.ds(0, _PASS)], s0),
              pltpu.async_copy(small.at[isa_r[p]], bufa.at[pl.ds(_PASS, _PASS)], s1),
              pltpu.async_copy(ent_out.at[ib_r[p]], bufb.at[pl.ds(0, _PASS)], s2),
              pltpu.async_copy(small.at[isb_r[p]], bufb.at[pl.ds(_PASS, _PASS)], s3),
              pltpu.async_copy(small.at[im_r[p]], bufm, s4)]
        for c in cs:
            c.wait()
        for g in range(_PASS // _L):
            s0_ = p * _PASS + g * _L
            lsl = pl.ds(s0_, _L)
            ra = vra[lsl]
            rb = vrb[lsl]
            rm = lax.iota(jnp.int32, _L) + (g * _L)
            ca = vca[lsl]
            cb = vcb[lsl]
            cm = vcm[lsl]
            pab, pam, pbm = zero, zero, zero
            for d in range(emb):
                va = plsc.load_gather(bufa, [ra, ca + d])
                vb = plsc.load_gather(bufb, [rb, cb + d])
                vmm = plsc.load_gather(bufm, [rm, cm + d])
                pab = pab + va * vb
                pam = pam + va * vmm
                pbm = pbm + vb * vmm
            dots[lsl] = pab - pam * pbm
    pltpu.sync_copy(dots, out.at[sl_all])


def _make_sc_dots(n, lanes, emb):
    ch = n // _NW
    mesh = plsc.VectorSubcoreMesh(core_axis_name="c", subcore_axis_name="s")
    return pl.kernel(
        functools.partial(_sc_dots_body, ch=ch, emb=emb),
        mesh=mesh,
        out_type=jax.ShapeDtypeStruct((n,), jnp.float32),
        scratch_types=[pltpu.VMEM((_PASS,), jnp.int32)] * 15
        + [pltpu.VMEM((ch,), jnp.int32)] * 5
        + [pltpu.VMEM((2 * _PASS, lanes), jnp.float32)] * 2
        + [pltpu.VMEM((_PASS, lanes), jnp.float32),
           pltpu.VMEM((ch,), jnp.float32)]
        + [pltpu.SemaphoreType.DMA] * 5,
        compiler_params=pltpu.CompilerParams(needs_layout_passes=False),
    )


def _log_sigmoid(x):
    return jnp.minimum(x, 0.0) - jnp.log(1.0 + jnp.exp(-jnp.abs(x)))


def _tc_loss_body(dp, dn, out):
    acc = _log_sigmoid(dp[...]) + jnp.sum(_log_sigmoid(-dn[...]),
                                          axis=1, keepdims=True)
    out[...] = -acc


def _prep(labels_in, labels_tgt, ne, rel, nq):
    ie = labels_in < ne
    te = labels_tgt < ne
    io = jnp.where(ie, labels_in, labels_in - ne).astype(jnp.int32)
    to = jnp.where(te, labels_tgt, labels_tgt - ne).astype(jnp.int32)
    case = jnp.where(ie & te, 0, jnp.where(ie & (~te), 1,
                     jnp.where((~ie) & te, 2, 3)))
    perm = jnp.argsort(case)
    io, to, ie, te, case = io[perm], to[perm], ie[perm], te[perm], case[perm]

    zg = (4 * rel) // nq
    gae = jnp.where(ie, io // nq, 0)
    gas = jnp.where(ie, 0, io // nq)
    gbe = jnp.where(te, to // nq, 0)
    gbs = jnp.where(te, 0, (rel + to) // nq)
    m_act = (case == 1) | (case == 2)
    row_m = jnp.where(case == 1, 2 * rel + to, 3 * rel + io)
    gm = jnp.where(m_act, row_m // nq, zg)
    sel_a = jnp.where(ie, 0, _PASS).astype(jnp.int32)
    sel_b = jnp.where(te, 0, _PASS).astype(jnp.int32)
    ca = (io % nq) * (128 // nq)
    cb = (to % nq) * (128 // nq)
    cm = jnp.where(m_act, (row_m % nq) * (128 // nq), 0)
    return gae, gas, gbe, gbs, gm, sel_a, sel_b, ca, cb, cm


def kernel(input_labels, pos_labels, neg_labels, ent_dic, reverse_dictionary,
           in_embed_ent, out_embed_ent, in_embed_rel, out_embed_rel,
           in_embed_map, out_embed_map):
    b = input_labels.shape[0]
    k = neg_labels.shape[0] // b
    emb = in_embed_ent.shape[1]
    ne = ent_dic.shape[0]
    rel = in_embed_rel.shape[0]
    n = b * (k + 1)
    nq = 128 // emb
    lanes = 128

    p = _prep(input_labels.reshape(-1), pos_labels.reshape(-1), ne, rel, nq)
    q = _prep(jnp.repeat(input_labels.reshape(-1), k),
              neg_labels.reshape(-1), ne, rel, nq)
    cat = [jnp.concatenate([pi, qi]) for pi, qi in zip(p, q)]
    gae, gas, gbe, gbs, gm, sel_a, sel_b, ca, cb, cm = cat
    pos_in_pass = (jnp.arange(n, dtype=jnp.int32) % _PASS)
    ra = sel_a + pos_in_pass
    rb = sel_b + pos_in_pass

    ent_in = in_embed_ent.reshape(-1, lanes)
    ent_out = out_embed_ent.reshape(-1, lanes)
    small = jnp.concatenate(
        [in_embed_rel, out_embed_rel, in_embed_map, out_embed_map,
         jnp.zeros((nq, emb), jnp.float32)], axis=0).reshape(-1, lanes)

    npass = (n // _NW) // _PASS
    g2 = lambda a: a.reshape(_NW * npass, _PASS)
    dots = _make_sc_dots(n, lanes, emb)(
        g2(gae), g2(gas), g2(gbe), g2(gbs), g2(gm), ra, rb, ca, cb, cm,
        ent_in, ent_out, small)

    loss2d = pl.pallas_call(
        _tc_loss_body,
        out_shape=jax.ShapeDtypeStruct((b, 1), jnp.float32),
    )(dots[:b].reshape(b, 1), dots[b:].reshape(b, k))
    return loss2d.reshape(b)


# speedup vs baseline: 4.4604x; 1.3149x over previous
import functools

import jax
import jax.numpy as jnp
from jax import lax
from jax.experimental import pallas as pl
from jax.experimental.pallas import tpu as pltpu
from jax.experimental.pallas import tpu_sc as plsc

_NW = 32
_SUB = 96


def _sc_gather_body(ia, ib, isa, isb, ism, ent_in, ent_out, small,
                    oa, ob, osa, osb, osm,
                    va, vb, vsa, vsb, vsm,
                    ra, rb, rsa, rsb, rsm,
                    s0, s1, s2, s3, s4, *, ch):
    wid = lax.axis_index("s") * 2 + lax.axis_index("c")
    base = wid * ch
    sl_all = pl.ds(base, ch)
    pltpu.sync_copy(ia.at[sl_all], va)
    pltpu.sync_copy(ib.at[sl_all], vb)
    pltpu.sync_copy(isa.at[sl_all], vsa)
    pltpu.sync_copy(isb.at[sl_all], vsb)
    pltpu.sync_copy(ism.at[sl_all], vsm)
    copies = []
    for k in range(ch // _SUB):
        sl = pl.ds(k * _SUB, _SUB)
        copies.append(pltpu.async_copy(ent_in.at[va.at[sl]], ra.at[sl], s0))
        copies.append(pltpu.async_copy(ent_out.at[vb.at[sl]], rb.at[sl], s1))
        copies.append(pltpu.async_copy(small.at[vsa.at[sl]], rsa.at[sl], s2))
        copies.append(pltpu.async_copy(small.at[vsb.at[sl]], rsb.at[sl], s3))
        copies.append(pltpu.async_copy(small.at[vsm.at[sl]], rsm.at[sl], s4))
    for c in copies:
        c.wait()
    pltpu.sync_copy(ra, oa.at[sl_all])
    pltpu.sync_copy(rb, ob.at[sl_all])
    pltpu.sync_copy(rsa, osa.at[sl_all])
    pltpu.sync_copy(rsb, osb.at[sl_all])
    pltpu.sync_copy(rsm, osm.at[sl_all])


def _make_sc_gather(n, emb):
    ch = n // _NW
    mesh = plsc.VectorSubcoreMesh(core_axis_name="c", subcore_axis_name="s")
    row = jax.ShapeDtypeStruct((n, emb), jnp.float32)
    return pl.kernel(
        functools.partial(_sc_gather_body, ch=ch),
        mesh=mesh,
        out_type=[row] * 5,
        scratch_types=[pltpu.VMEM((ch,), jnp.int32)] * 5
        + [pltpu.VMEM((ch, emb), jnp.float32)] * 5
        + [pltpu.SemaphoreType.DMA] * 5,
        compiler_params=pltpu.CompilerParams(use_tc_tiling_on_sc=False),
    )


def _log_sigmoid(x):
    return jnp.minimum(x, 0.0) - jnp.log(1.0 + jnp.exp(-jnp.abs(x)))


def _tc_loss_body(ape, bpe, aps, bps, mp, ane, bne, ans, bns, mn,
                  map_, mbp, man, mbn, out, *, emb, k):
    ap = ape[...] * map_[...] + aps[...]
    bp = bpe[...] * mbp[...] + bps[...]
    mpv = mp[...]
    dp = (jnp.sum(ap * bp, axis=1, keepdims=True)
          - jnp.sum(ap * mpv, axis=1, keepdims=True)
          * jnp.sum(bp * mpv, axis=1, keepdims=True))
    acc = _log_sigmoid(dp)
    ane_v, bne_v, ans_v, bns_v, mn_v = ane[...], bne[...], ans[...], bns[...], mn[...]
    man_v, mbn_v = man[...], mbn[...]
    for j in range(k):
        sl = slice(j * emb, (j + 1) * emb)
        aj = ane_v[:, sl] * man_v[:, j:j + 1] + ans_v[:, sl]
        bj = bne_v[:, sl] * mbn_v[:, j:j + 1] + bns_v[:, sl]
        mj = mn_v[:, sl]
        dnj = (jnp.sum(aj * bj, axis=1, keepdims=True)
               - jnp.sum(aj * mj, axis=1, keepdims=True)
               * jnp.sum(bj * mj, axis=1, keepdims=True))
        acc = acc + _log_sigmoid(-dnj)
    out[...] = -acc


def _prep(labels_in, labels_tgt, ne, rel):
    ie = labels_in < ne
    te = labels_tgt < ne
    io = jnp.where(ie, labels_in, labels_in - ne).astype(jnp.int32)
    to = jnp.where(te, labels_tgt, labels_tgt - ne).astype(jnp.int32)
    case = jnp.where(ie & te, 0, jnp.where(ie & (~te), 1,
                     jnp.where((~ie) & te, 2, 3)))
    perm = jnp.argsort(case)
    io, to, ie, te, case = io[perm], to[perm], ie[perm], te[perm], case[perm]
    zrow = 4 * rel
    idx_ae = jnp.where(ie, io, 0)
    idx_be = jnp.where(te, to, 0)
    idx_sa = jnp.where(ie, zrow, io)
    idx_sb = jnp.where(te, zrow, rel + to)
    idx_sm = jnp.where(case == 1, 2 * rel + to,
                       jnp.where(case == 2, 3 * rel + io, zrow))
    return idx_ae, idx_be, idx_sa, idx_sb, idx_sm, ie, te


def kernel(input_labels, pos_labels, neg_labels, ent_dic, reverse_dictionary,
           in_embed_ent, out_embed_ent, in_embed_rel, out_embed_rel,
           in_embed_map, out_embed_map):
    b = input_labels.shape[0]
    k = neg_labels.shape[0] // b
    emb = in_embed_ent.shape[1]
    ne = ent_dic.shape[0]
    rel = in_embed_rel.shape[0]
    n = b * (k + 1)

    p = _prep(input_labels.reshape(-1), pos_labels.reshape(-1), ne, rel)
    q = _prep(jnp.repeat(input_labels.reshape(-1), k), neg_labels.reshape(-1),
              ne, rel)
    idxs = [jnp.concatenate([pi, qi]) for pi, qi in zip(p[:5], q[:5])]

    small = jnp.concatenate(
        [in_embed_rel, out_embed_rel, in_embed_map, out_embed_map,
         jnp.zeros((1, emb), jnp.float32)], axis=0)

    oa, ob, osa, osb, osm = _make_sc_gather(n, emb)(
        *idxs, in_embed_ent, out_embed_ent, small)

    f32 = jnp.float32
    map_p = p[5].astype(f32)[:, None]
    mbp = p[6].astype(f32)[:, None]
    man = q[5].astype(f32).reshape(b, k)
    mbn = q[6].astype(f32).reshape(b, k)

    loss2d = pl.pallas_call(
        functools.partial(_tc_loss_body, emb=emb, k=k),
        out_shape=jax.ShapeDtypeStruct((b, 1), f32),
    )(oa[:b], ob[:b], osa[:b], osb[:b], osm[:b],
      oa[b:].reshape(b, k * emb), ob[b:].reshape(b, k * emb),
      osa[b:].reshape(b, k * emb), osb[b:].reshape(b, k * emb),
      osm[b:].reshape(b, k * emb),
      map_p, mbp, man, mbn)
    return loss2d.reshape(b)
